# baseline (speedup 1.0000x reference)
import functools

import jax
import jax.numpy as jnp
from jax import lax
from jax.experimental import pallas as pl
from jax.experimental.pallas import tpu as pltpu
from jax.experimental.pallas import tpu_sc as plsc

_NC = 2
_NS = 16
_NW = _NC * _NS
_L = 16
_B = 128
_BLK = 2048
_G = 64


def _sc_mesh():
    return plsc.VectorSubcoreMesh(
        core_axis_name="c", subcore_axis_name="s",
        num_cores=_NC, num_subcores=_NS)


def _make_deg(Npad, Epad):
    EW = Epad // _NW
    nchunks = EW // _B
    RPT = Npad // _NS

    @functools.partial(
        pl.kernel,
        out_type=jax.ShapeDtypeStruct((_NC, Npad), jnp.float32),
        mesh=_sc_mesh(),
        scratch_types=[
            pltpu.VMEM((nchunks, _B), jnp.int32),
            pltpu.VMEM((_B,), jnp.float32),
            pltpu.VMEM((RPT,), jnp.float32),
            pltpu.VMEM_SHARED((Npad,), jnp.float32),
            pltpu.SemaphoreType.DMA,
            pltpu.SemaphoreType.DMA,
        ],
    )
    def deg(dst_hbm, out_hbm, dsts_v, ones_v, zeros_v, acc, sem0, sem1):
        c = lax.axis_index("c")
        s = lax.axis_index("s")
        wid = s * _NC + c
        for i in range(_B // _L):
            ones_v[pl.ds(i * _L, _L)] = jnp.full((_L,), 1.0, jnp.float32)
        for i in range(RPT // _L):
            zeros_v[pl.ds(i * _L, _L)] = jnp.zeros((_L,), jnp.float32)
        pltpu.sync_copy(dst_hbm.at[wid], dsts_v)
        pltpu.sync_copy(zeros_v, acc.at[pl.ds(s * RPT, RPT)])
        plsc.subcore_barrier()

        def body(jp, carry):
            j0 = 2 * jp

            @pl.when(jp > 0)
            def _():
                pltpu.make_async_copy(ones_v, acc.at[dsts_v.at[0]],
                                      sem0).wait()
            pltpu.async_copy(ones_v, acc.at[dsts_v.at[j0]], sem0, add=True)

            @pl.when(jp > 0)
            def _():
                pltpu.make_async_copy(ones_v, acc.at[dsts_v.at[0]],
                                      sem1).wait()
            pltpu.async_copy(ones_v, acc.at[dsts_v.at[j0 + 1]], sem1,
                             add=True)
            return carry

        lax.fori_loop(0, nchunks // 2, body, 0)
        pltpu.make_async_copy(ones_v, acc.at[dsts_v.at[0]], sem0).wait()
        pltpu.make_async_copy(ones_v, acc.at[dsts_v.at[0]], sem1).wait()
        plsc.subcore_barrier()
        pltpu.sync_copy(acc.at[pl.ds(s * RPT, RPT)],
                        out_hbm.at[c, pl.ds(s * RPT, RPT)])

    return deg


def _make_agg(Npad, D, Epad):
    EW = Epad // _NW
    nchunks = EW // _B
    RPT = Npad // _NS

    @functools.partial(
        pl.kernel,
        out_type=jax.ShapeDtypeStruct((_NC, Npad, D), jnp.float32),
        mesh=_sc_mesh(),
        scratch_types=[
            pltpu.VMEM((_B,), jnp.int32),
            pltpu.VMEM((_B,), jnp.int32),
            pltpu.VMEM((_B,), jnp.int32),
            pltpu.VMEM((_B,), jnp.int32),
            pltpu.VMEM((_B, D), jnp.float32),
            pltpu.VMEM((_B, D), jnp.float32),
            pltpu.VMEM_SHARED((Npad, D), jnp.float32),
            pltpu.SemaphoreType.DMA,
            pltpu.SemaphoreType.DMA,
            pltpu.SemaphoreType.DMA,
            pltpu.SemaphoreType.DMA,
            pltpu.SemaphoreType.DMA,
            pltpu.SemaphoreType.DMA,
        ],
    )
    def agg(p_hbm, src_hbm, dst_hbm, out_hbm, s0, s1, d0, d1, rows0, rows1,
            acc, gsem0, gsem1, ssem0, ssem1, dsem0, dsem1):
        c = lax.axis_index("c")
        s = lax.axis_index("s")
        wid = s * _NC + c
        pltpu.sync_copy(p_hbm.at[pl.ds(s * RPT, RPT)],
                        acc.at[pl.ds(s * RPT, RPT)])
        plsc.subcore_barrier()

        pltpu.sync_copy(src_hbm.at[wid, 0], s0)
        pltpu.sync_copy(src_hbm.at[wid, 1], s1)
        pltpu.sync_copy(dst_hbm.at[wid, 0], d0)
        pltpu.sync_copy(dst_hbm.at[wid, 1], d1)
        pltpu.async_copy(p_hbm.at[s0], rows0, gsem0)
        pltpu.async_copy(p_hbm.at[s1], rows1, gsem1)

        def half(jp, j, sv, dv, rows, gsem, ssem, dsem):
            @pl.when(jp > 0)
            def _():
                pltpu.make_async_copy(dst_hbm.at[wid, 0], dv, dsem).wait()
            pltpu.make_async_copy(p_hbm.at[sv], rows, gsem).wait()

            @pl.when(j + 2 < nchunks)
            def _():
                pltpu.async_copy(src_hbm.at[wid, j + 2], sv, ssem)
            pltpu.sync_copy(rows, acc.at[dv], add=True)

            @pl.when(j + 2 < nchunks)
            def _():
                pltpu.async_copy(dst_hbm.at[wid, j + 2], dv, dsem)
                pltpu.make_async_copy(src_hbm.at[wid, 0], sv, ssem).wait()
                pltpu.async_copy(p_hbm.at[sv], rows, gsem)

        def body(jp, carry):
            j0 = 2 * jp
            half(jp, j0, s0, d0, rows0, gsem0, ssem0, dsem0)
            half(jp, j0 + 1, s1, d1, rows1, gsem1, ssem1, dsem1)
            return carry

        lax.fori_loop(0, nchunks // 2, body, 0)
        plsc.subcore_barrier()
        pltpu.sync_copy(acc.at[pl.ds(s * RPT, RPT)],
                        out_hbm.at[c, pl.ds(s * RPT, RPT)])

    return agg


def _mm_scale(x_p, W, degT, N):
    Npad, D = x_p.shape
    H = W.shape[1]
    grid = Npad // _BLK

    def body(x_ref, w_ref, deg_ref, out_ref):
        i = pl.program_id(0)
        d = jnp.sum(deg_ref[...], axis=1, keepdims=True) + 1.0
        rows = lax.broadcasted_iota(jnp.int32, (_BLK, 1), 0) + i * _BLK
        dinv = jnp.where(rows < N, lax.rsqrt(d), 0.0)
        out_ref[...] = jnp.dot(x_ref[...], w_ref[...],
                               preferred_element_type=jnp.float32) * dinv

    return pl.pallas_call(
        body,
        grid=(grid,),
        in_specs=[
            pl.BlockSpec((_BLK, D), lambda i: (i, 0)),
            pl.BlockSpec((D, H), lambda i: (0, 0)),
            pl.BlockSpec((_BLK, _NC), lambda i: (i, 0)),
        ],
        out_specs=pl.BlockSpec((_BLK, H), lambda i: (i, 0)),
        out_shape=jax.ShapeDtypeStruct((Npad, H), jnp.float32),
    )(x_p, W, degT)


def _layer_mm(aggp, p_prev, degT, b, W, N):
    Npad, H = p_prev.shape
    grid = Npad // _BLK

    def body(a_ref, p_ref, deg_ref, b_ref, w_ref, out_ref):
        i = pl.program_id(0)
        d = jnp.sum(deg_ref[...], axis=1, keepdims=True) + 1.0
        rows = lax.broadcasted_iota(jnp.int32, (_BLK, 1), 0) + i * _BLK
        dinv = jnp.where(rows < N, lax.rsqrt(d), 0.0)
        h = jnp.maximum(dinv * (a_ref[0] + a_ref[1] - p_ref[...]) + b_ref[...],
                        0.0)
        out_ref[...] = jnp.dot(h, w_ref[...],
                               preferred_element_type=jnp.float32) * dinv

    return pl.pallas_call(
        body,
        grid=(grid,),
        in_specs=[
            pl.BlockSpec((_NC, _BLK, H), lambda i: (0, i, 0)),
            pl.BlockSpec((_BLK, H), lambda i: (i, 0)),
            pl.BlockSpec((_BLK, _NC), lambda i: (i, 0)),
            pl.BlockSpec((1, H), lambda i: (0, 0)),
            pl.BlockSpec((H, H), lambda i: (0, 0)),
        ],
        out_specs=pl.BlockSpec((_BLK, H), lambda i: (i, 0)),
        out_shape=jax.ShapeDtypeStruct((Npad, H), jnp.float32),
    )(aggp, p_prev, degT, b, W)


def _final(aggp, p_prev, degT, b, batch_p, Wc, bc, N):
    Npad, H = p_prev.shape
    grid = Npad // _BLK

    def body(a_ref, p_ref, deg_ref, b_ref, bt_ref, wc_ref, bc_ref, y_ref,
             sums, cnts):
        i = pl.program_id(0)

        @pl.when(i == 0)
        def _():
            sums[...] = jnp.zeros_like(sums)
            cnts[...] = jnp.zeros_like(cnts)

        d = jnp.sum(deg_ref[...], axis=1, keepdims=True) + 1.0
        rows = lax.broadcasted_iota(jnp.int32, (_BLK, 1), 0) + i * _BLK
        valid = rows < N
        dinv = jnp.where(valid, lax.rsqrt(d), 0.0)
        h = jnp.maximum(dinv * (a_ref[0] + a_ref[1] - p_ref[...]) + b_ref[...],
                        0.0)
        M = jnp.where(
            (bt_ref[...] == lax.broadcasted_iota(jnp.int32, (_BLK, _G), 1))
            & valid, 1.0, 0.0)
        sums[...] += lax.dot_general(M, h, (((0,), (0,)), ((), ())),
                                     preferred_element_type=jnp.float32)
        cnts[...] += lax.dot_general(M, jnp.ones((_BLK, H), jnp.float32),
                                     (((0,), (0,)), ((), ())),
                                     preferred_element_type=jnp.float32)

        @pl.when(i == grid - 1)
        def _():
            pooled = sums[...] / jnp.maximum(cnts[...], 1.0)
            y_ref[...] = jnp.dot(pooled, wc_ref[...],
                                 preferred_element_type=jnp.float32) + bc_ref[...]

    return pl.pallas_call(
        body,
        grid=(grid,),
        in_specs=[
            pl.BlockSpec((_NC, _BLK, H), lambda i: (0, i, 0)),
            pl.BlockSpec((_BLK, H), lambda i: (i, 0)),
            pl.BlockSpec((_BLK, _NC), lambda i: (i, 0)),
            pl.BlockSpec((1, H), lambda i: (0, 0)),
            pl.BlockSpec((_BLK, 1), lambda i: (i, 0)),
            pl.BlockSpec((H, 1), lambda i: (0, 0)),
            pl.BlockSpec((1, 1), lambda i: (0, 0)),
        ],
        out_specs=pl.BlockSpec((_G, 1), lambda i: (0, 0)),
        out_shape=jax.ShapeDtypeStruct((_G, 1), jnp.float32),
        scratch_shapes=[
            pltpu.VMEM((_G, H), jnp.float32),
            pltpu.VMEM((_G, H), jnp.float32),
        ],
    )(aggp, p_prev, degT, b, batch_p, Wc, bc)


def kernel(x, edge_index, batch, W1, b1, W2, b2, Wc, bc):
    N, D = x.shape
    H = W1.shape[1]
    E = edge_index.shape[1]

    Npad = ((N + 1 + _BLK - 1) // _BLK) * _BLK
    EW = ((E // _NW + 2 * _B - 1) // (2 * _B)) * (2 * _B)
    Epad = EW * _NW
    nchunks = EW // _B

    src = jnp.concatenate(
        [edge_index[0], jnp.full((Epad - E,), N, jnp.int32)]
    ).reshape(_NW, nchunks, _B)
    dst = jnp.concatenate(
        [edge_index[1], jnp.full((Epad - E,), N, jnp.int32)]
    ).reshape(_NW, nchunks, _B)
    x_p = jnp.zeros((Npad, D), jnp.float32).at[:N].set(x)
    batch_p = jnp.full((Npad, 1), _G, jnp.int32).at[:N, 0].set(batch)

    deg_parts = _make_deg(Npad, Epad)(dst)
    degT = deg_parts.T

    agg = _make_agg(Npad, D, Epad)

    p1 = _mm_scale(x_p, W1, degT, N)
    a1 = agg(p1, src, dst)
    p2 = _layer_mm(a1, p1, degT, b1.reshape(1, H), W2, N)
    a2 = agg(p2, src, dst)
    y = _final(a2, p2, degT, b2.reshape(1, H), batch_p,
               Wc, bc.reshape(1, 1), N)
    return y


# speedup vs baseline: 1.0968x; 1.0968x over previous
import functools

import jax
import jax.numpy as jnp
from jax import lax
from jax.experimental import pallas as pl
from jax.experimental.pallas import tpu as pltpu
from jax.experimental.pallas import tpu_sc as plsc

_NC = 2
_NS = 16
_NW = _NC * _NS
_L = 16
_B = 128
_BLK = 2048
_G = 64
_F0 = 0.81


def _sc_mesh():
    return plsc.VectorSubcoreMesh(
        core_axis_name="c", subcore_axis_name="s",
        num_cores=_NC, num_subcores=_NS)


def _make_deg(Npad, CNT0, CNT1):
    CMAX = max(CNT0, CNT1)
    RPT = Npad // _NS

    @functools.partial(
        pl.kernel,
        out_type=jax.ShapeDtypeStruct((_NC, Npad), jnp.float32),
        mesh=_sc_mesh(),
        scratch_types=[
            pltpu.VMEM((CMAX, _B), jnp.int32),
            pltpu.VMEM((_B,), jnp.float32),
            pltpu.VMEM((RPT,), jnp.float32),
            pltpu.VMEM_SHARED((Npad,), jnp.float32),
            pltpu.SemaphoreType.DMA,
            pltpu.SemaphoreType.DMA,
        ],
    )
    def deg(dst_hbm, out_hbm, dsts_v, ones_v, zeros_v, acc, sem0, sem1):
        c = lax.axis_index("c")
        s = lax.axis_index("s")
        my_cnt = jnp.where(c == 0, CNT0, CNT1)
        chunk0 = pl.multiple_of(
            jnp.where(c == 0, s * CNT0, _NS * CNT0 + s * CNT1), 8)
        for i in range(_B // _L):
            ones_v[pl.ds(i * _L, _L)] = jnp.full((_L,), 1.0, jnp.float32)
        for i in range(RPT // _L):
            zeros_v[pl.ds(i * _L, _L)] = jnp.zeros((_L,), jnp.float32)

        @pl.when(c == 0)
        def _():
            pltpu.sync_copy(dst_hbm.at[pl.ds(chunk0, CNT0)],
                            dsts_v.at[pl.ds(0, CNT0)])

        @pl.when(c != 0)
        def _():
            pltpu.sync_copy(dst_hbm.at[pl.ds(chunk0, CNT1)],
                            dsts_v.at[pl.ds(0, CNT1)])
        pltpu.sync_copy(zeros_v, acc.at[pl.ds(s * RPT, RPT)])
        plsc.subcore_barrier()

        def body(jp, carry):
            j0 = 2 * jp

            @pl.when(jp > 0)
            def _():
                pltpu.make_async_copy(ones_v, acc.at[dsts_v.at[0]],
                                      sem0).wait()
            pltpu.async_copy(ones_v, acc.at[dsts_v.at[j0]], sem0, add=True)

            @pl.when(jp > 0)
            def _():
                pltpu.make_async_copy(ones_v, acc.at[dsts_v.at[0]],
                                      sem1).wait()
            pltpu.async_copy(ones_v, acc.at[dsts_v.at[j0 + 1]], sem1,
                             add=True)
            return carry

        lax.fori_loop(0, my_cnt // 2, body, 0)
        pltpu.make_async_copy(ones_v, acc.at[dsts_v.at[0]], sem0).wait()
        pltpu.make_async_copy(ones_v, acc.at[dsts_v.at[0]], sem1).wait()
        plsc.subcore_barrier()
        pltpu.sync_copy(acc.at[pl.ds(s * RPT, RPT)],
                        out_hbm.at[c, pl.ds(s * RPT, RPT)])

    return deg


def _make_agg(Npad, D, CNT0, CNT1):
    RPT = Npad // _NS

    @functools.partial(
        pl.kernel,
        out_type=jax.ShapeDtypeStruct((_NC, Npad, D), jnp.float32),
        mesh=_sc_mesh(),
        scratch_types=[
            pltpu.VMEM((_B,), jnp.int32),
            pltpu.VMEM((_B,), jnp.int32),
            pltpu.VMEM((_B,), jnp.int32),
            pltpu.VMEM((_B,), jnp.int32),
            pltpu.VMEM((_B, D), jnp.float32),
            pltpu.VMEM((_B, D), jnp.float32),
            pltpu.VMEM_SHARED((Npad, D), jnp.float32),
            pltpu.SemaphoreType.DMA,
            pltpu.SemaphoreType.DMA,
            pltpu.SemaphoreType.DMA,
            pltpu.SemaphoreType.DMA,
            pltpu.SemaphoreType.DMA,
            pltpu.SemaphoreType.DMA,
        ],
    )
    def agg(p_hbm, src_hbm, dst_hbm, out_hbm, s0, s1, d0, d1, rows0, rows1,
            acc, gsem0, gsem1, ssem0, ssem1, dsem0, dsem1):
        c = lax.axis_index("c")
        s = lax.axis_index("s")
        my_cnt = jnp.where(c == 0, CNT0, CNT1)
        chunk0 = jnp.where(c == 0, s * CNT0, _NS * CNT0 + s * CNT1)
        pltpu.sync_copy(p_hbm.at[pl.ds(s * RPT, RPT)],
                        acc.at[pl.ds(s * RPT, RPT)])
        plsc.subcore_barrier()

        pltpu.sync_copy(src_hbm.at[chunk0], s0)
        pltpu.sync_copy(src_hbm.at[chunk0 + 1], s1)
        pltpu.sync_copy(dst_hbm.at[chunk0], d0)
        pltpu.sync_copy(dst_hbm.at[chunk0 + 1], d1)
        pltpu.async_copy(p_hbm.at[s0], rows0, gsem0)
        pltpu.async_copy(p_hbm.at[s1], rows1, gsem1)

        def half(jp, off, sv, dv, rows, gsem, ssem, dsem):
            j = 2 * jp + off

            @pl.when(jp > 0)
            def _():
                pltpu.make_async_copy(dst_hbm.at[chunk0], dv, dsem).wait()
            pltpu.make_async_copy(p_hbm.at[sv], rows, gsem).wait()

            @pl.when(j + 2 < my_cnt)
            def _():
                pltpu.async_copy(src_hbm.at[chunk0 + j + 2], sv, ssem)
            pltpu.sync_copy(rows, acc.at[dv], add=True)

            @pl.when(j + 2 < my_cnt)
            def _():
                pltpu.async_copy(dst_hbm.at[chunk0 + j + 2], dv, dsem)
                pltpu.make_async_copy(src_hbm.at[chunk0], sv, ssem).wait()
                pltpu.async_copy(p_hbm.at[sv], rows, gsem)

        def body(jp, carry):
            half(jp, 0, s0, d0, rows0, gsem0, ssem0, dsem0)
            half(jp, 1, s1, d1, rows1, gsem1, ssem1, dsem1)
            return carry

        lax.fori_loop(0, my_cnt // 2, body, 0)
        plsc.subcore_barrier()
        pltpu.sync_copy(acc.at[pl.ds(s * RPT, RPT)],
                        out_hbm.at[c, pl.ds(s * RPT, RPT)])

    return agg


def _mm_scale(x_p, W, degT, N):
    Npad, D = x_p.shape
    H = W.shape[1]
    grid = Npad // _BLK

    def body(x_ref, w_ref, deg_ref, out_ref):
        i = pl.program_id(0)
        d = jnp.sum(deg_ref[...], axis=1, keepdims=True) + 1.0
        rows = lax.broadcasted_iota(jnp.int32, (_BLK, 1), 0) + i * _BLK
        dinv = jnp.where(rows < N, lax.rsqrt(d), 0.0)
        out_ref[...] = jnp.dot(x_ref[...], w_ref[...],
                               preferred_element_type=jnp.float32) * dinv

    return pl.pallas_call(
        body,
        grid=(grid,),
        in_specs=[
            pl.BlockSpec((_BLK, D), lambda i: (i, 0)),
            pl.BlockSpec((D, H), lambda i: (0, 0)),
            pl.BlockSpec((_BLK, _NC), lambda i: (i, 0)),
        ],
        out_specs=pl.BlockSpec((_BLK, H), lambda i: (i, 0)),
        out_shape=jax.ShapeDtypeStruct((Npad, H), jnp.float32),
    )(x_p, W, degT)


def _layer_mm(aggp, p_prev, degT, b, W, N):
    Npad, H = p_prev.shape
    grid = Npad // _BLK

    def body(a_ref, p_ref, deg_ref, b_ref, w_ref, out_ref):
        i = pl.program_id(0)
        d = jnp.sum(deg_ref[...], axis=1, keepdims=True) + 1.0
        rows = lax.broadcasted_iota(jnp.int32, (_BLK, 1), 0) + i * _BLK
        dinv = jnp.where(rows < N, lax.rsqrt(d), 0.0)
        h = jnp.maximum(dinv * (a_ref[0] + a_ref[1] - p_ref[...]) + b_ref[...],
                        0.0)
        out_ref[...] = jnp.dot(h, w_ref[...],
                               preferred_element_type=jnp.float32) * dinv

    return pl.pallas_call(
        body,
        grid=(grid,),
        in_specs=[
            pl.BlockSpec((_NC, _BLK, H), lambda i: (0, i, 0)),
            pl.BlockSpec((_BLK, H), lambda i: (i, 0)),
            pl.BlockSpec((_BLK, _NC), lambda i: (i, 0)),
            pl.BlockSpec((1, H), lambda i: (0, 0)),
            pl.BlockSpec((H, H), lambda i: (0, 0)),
        ],
        out_specs=pl.BlockSpec((_BLK, H), lambda i: (i, 0)),
        out_shape=jax.ShapeDtypeStruct((Npad, H), jnp.float32),
    )(aggp, p_prev, degT, b, W)


def _final(aggp, p_prev, degT, b, batch_p, Wc, bc, N):
    Npad, H = p_prev.shape
    grid = Npad // _BLK

    def body(a_ref, p_ref, deg_ref, b_ref, bt_ref, wc_ref, bc_ref, y_ref,
             sums, cnts):
        i = pl.program_id(0)

        @pl.when(i == 0)
        def _():
            sums[...] = jnp.zeros_like(sums)
            cnts[...] = jnp.zeros_like(cnts)

        d = jnp.sum(deg_ref[...], axis=1, keepdims=True) + 1.0
        rows = lax.broadcasted_iota(jnp.int32, (_BLK, 1), 0) + i * _BLK
        valid = rows < N
        dinv = jnp.where(valid, lax.rsqrt(d), 0.0)
        h = jnp.maximum(dinv * (a_ref[0] + a_ref[1] - p_ref[...]) + b_ref[...],
                        0.0)
        M = jnp.where(
            (bt_ref[...] == lax.broadcasted_iota(jnp.int32, (_BLK, _G), 1))
            & valid, 1.0, 0.0)
        sums[...] += lax.dot_general(M, h, (((0,), (0,)), ((), ())),
                                     preferred_element_type=jnp.float32)
        cnts[...] += lax.dot_general(M, jnp.ones((_BLK, H), jnp.float32),
                                     (((0,), (0,)), ((), ())),
                                     preferred_element_type=jnp.float32)

        @pl.when(i == grid - 1)
        def _():
            pooled = sums[...] / jnp.maximum(cnts[...], 1.0)
            y_ref[...] = jnp.dot(pooled, wc_ref[...],
                                 preferred_element_type=jnp.float32) + bc_ref[...]

    return pl.pallas_call(
        body,
        grid=(grid,),
        in_specs=[
            pl.BlockSpec((_NC, _BLK, H), lambda i: (0, i, 0)),
            pl.BlockSpec((_BLK, H), lambda i: (i, 0)),
            pl.BlockSpec((_BLK, _NC), lambda i: (i, 0)),
            pl.BlockSpec((1, H), lambda i: (0, 0)),
            pl.BlockSpec((_BLK, 1), lambda i: (i, 0)),
            pl.BlockSpec((H, 1), lambda i: (0, 0)),
            pl.BlockSpec((1, 1), lambda i: (0, 0)),
        ],
        out_specs=pl.BlockSpec((_G, 1), lambda i: (0, 0)),
        out_shape=jax.ShapeDtypeStruct((_G, 1), jnp.float32),
        scratch_shapes=[
            pltpu.VMEM((_G, H), jnp.float32),
            pltpu.VMEM((_G, H), jnp.float32),
        ],
    )(aggp, p_prev, degT, b, batch_p, Wc, bc)


def kernel(x, edge_index, batch, W1, b1, W2, b2, Wc, bc):
    N, D = x.shape
    H = W1.shape[1]
    E = edge_index.shape[1]

    Npad = ((N + 1 + _BLK - 1) // _BLK) * _BLK
    S = (E + _NS * _B - 1) // (_NS * _B)
    cnt0 = max(8, (int(S * _F0) + 7) // 8 * 8)
    cnt1 = max(8, (S - cnt0 + 7) // 8 * 8)
    Epad = _NS * (cnt0 + cnt1) * _B
    nchunks = Epad // _B

    src = jnp.concatenate(
        [edge_index[0], jnp.full((Epad - E,), N, jnp.int32)]
    ).reshape(nchunks, _B)
    dst = jnp.concatenate(
        [edge_index[1], jnp.full((Epad - E,), N, jnp.int32)]
    ).reshape(nchunks, _B)
    x_p = jnp.zeros((Npad, D), jnp.float32).at[:N].set(x)
    batch_p = jnp.full((Npad, 1), _G, jnp.int32).at[:N, 0].set(batch)

    deg_parts = _make_deg(Npad, cnt0, cnt1)(dst)
    degT = deg_parts.T

    agg = _make_agg(Npad, D, cnt0, cnt1)

    p1 = _mm_scale(x_p, W1, degT, N)
    a1 = agg(p1, src, dst)
    p2 = _layer_mm(a1, p1, degT, b1.reshape(1, H), W2, N)
    a2 = agg(p2, src, dst)
    y = _final(a2, p2, degT, b2.reshape(1, H), batch_p,
               Wc, bc.reshape(1, 1), N)
    return y
